# baseline (speedup 1.0000x reference)
import functools

import jax
import jax.numpy as jnp
from jax import lax
from jax.experimental import pallas as pl
from jax.experimental.pallas import tpu as pltpu
from jax.experimental.pallas import tpu_sc as plsc

_LANES = 16


@functools.lru_cache(maxsize=None)
def _build(num_ent2, num_rel2, dim, batch):
    info = plsc.get_sparse_core_info()
    nc, ns = info.num_cores, info.num_subcores
    nw = nc * ns
    assert batch % (8 * nw) == 0 and dim % _LANES == 0
    bpw = batch // nw
    cs = min(bpw, 128)
    nchunks = bpw // cs
    nvec = dim // _LANES

    mesh = plsc.VectorSubcoreMesh(core_axis_name="c", subcore_axis_name="s")

    @functools.partial(
        pl.kernel,
        mesh=mesh,
        out_type=jax.ShapeDtypeStruct((batch, dim), jnp.float32),
        compiler_params=pltpu.CompilerParams(needs_layout_passes=False),
        scratch_types=[
            pltpu.VMEM((bpw,), jnp.int32),
            pltpu.VMEM((bpw,), jnp.int32),
            pltpu.VMEM((bpw,), jnp.int32),
            pltpu.VMEM((bpw,), jnp.int32),
            pltpu.VMEM((bpw,), jnp.int32),
            pltpu.VMEM((bpw,), jnp.int32),
            pltpu.VMEM((cs, 2 * dim), jnp.float32),
            pltpu.VMEM((cs, 2 * dim), jnp.float32),
            pltpu.VMEM((cs, 2 * dim), jnp.float32),
            pltpu.VMEM((cs, dim), jnp.float32),
            pltpu.SemaphoreType.DMA,
            pltpu.SemaphoreType.DMA,
            pltpu.SemaphoreType.DMA,
        ],
    )
    def k(ent_hbm, rel_hbm, h_hbm, t_hbm, r_hbm, out_hbm,
          hrow, trow, rrow, hoff, toff, roff,
          hbuf, tbuf, rbuf, obuf, sem_h, sem_t, sem_r):
        wid = lax.axis_index("s") * nc + lax.axis_index("c")
        base = wid * bpw
        pltpu.sync_copy(h_hbm.at[pl.ds(base, bpw)], hrow)
        pltpu.sync_copy(t_hbm.at[pl.ds(base, bpw)], trow)
        pltpu.sync_copy(r_hbm.at[pl.ds(base, bpw)], rrow)

        def prep(j, carry):
            sl = pl.ds(j * _LANES, _LANES)
            for idx_ref, off_ref in ((hrow, hoff), (trow, toff), (rrow, roff)):
                v = idx_ref[sl]
                off_ref[sl] = (v & 1) * dim
                idx_ref[sl] = lax.shift_right_logical(v, 1)
            return carry

        lax.fori_loop(0, bpw // _LANES, prep, 0)

        lanes = lax.iota(jnp.int32, _LANES)
        for g in range(nchunks):
            gsl = pl.ds(g * cs, cs)
            ch = pltpu.async_copy(ent_hbm.at[hrow.at[gsl]], hbuf, sem_h)
            ct = pltpu.async_copy(ent_hbm.at[trow.at[gsl]], tbuf, sem_t)
            cr = pltpu.async_copy(rel_hbm.at[rrow.at[gsl]], rbuf, sem_r)
            ch.wait()
            ct.wait()
            cr.wait()

            def combine(i, carry):
                isplat = lax.broadcast(i, (_LANES,))
                gi = lax.broadcast(g * cs + i, (_LANES,))
                oh = plsc.load_gather(hoff, [gi])
                ot = plsc.load_gather(toff, [gi])
                orr = plsc.load_gather(roff, [gi])
                for c in range(nvec):
                    col = lanes + (c * _LANES)
                    hv = plsc.load_gather(hbuf, [isplat, oh + col])
                    tv = plsc.load_gather(tbuf, [isplat, ot + col])
                    rv = plsc.load_gather(rbuf, [isplat, orr + col])
                    obuf[i, pl.ds(c * _LANES, _LANES)] = hv + rv - tv
                return carry

            lax.fori_loop(0, cs, combine, 0)
            pltpu.sync_copy(obuf, out_hbm.at[pl.ds(base + g * cs, cs)])

    return k


def kernel(ent_table, rel_table, h_list, t_list, r_list):
    num_ent, dim = ent_table.shape
    num_rel = rel_table.shape[0]
    batch = h_list.shape[0]
    assert num_ent % 2 == 0 and num_rel % 2 == 0
    ent2 = ent_table.reshape(num_ent // 2, 2 * dim)
    rel2 = rel_table.reshape(num_rel // 2, 2 * dim)
    k = _build(num_ent // 2, num_rel // 2, dim, batch)
    return k(ent2, rel2,
             h_list.astype(jnp.int32), t_list.astype(jnp.int32),
             r_list.astype(jnp.int32))
